# baseline (speedup 1.0000x reference)
import functools

import jax
import jax.numpy as jnp
from jax import lax
from jax.experimental import pallas as pl
from jax.experimental.pallas import tpu as pltpu
from jax.experimental.pallas import tpu_sc as plsc

B, C, H, W = 8, 128, 224, 224
HW = H * W
HID = 64
K = 30
KP = 32
HCH = H // 4
CH = HW // 4
WV = W // 16
T0 = 0.995
T1 = 0.5
NEG = -2.0
BIGI = 1 << 30
CG = C // 4


def _extract_topk(src_v, src_i, dst_v, dst_i, nvregs):
    iota = lax.iota(jnp.int32, 16)
    bigv = jnp.full((16,), BIGI, jnp.int32)
    negv16 = jnp.full((16,), NEG, jnp.float32)

    def body(k, _):
        def mx(j, carry):
            bv, bi_ = carry
            v = src_v[pl.ds(j * 16, 16)]
            ii = src_i[pl.ds(j * 16, 16)]
            gt = v > bv
            eq = v == bv
            bi_ = jnp.where(gt, ii, jnp.where(eq, jnp.minimum(bi_, ii), bi_))
            return jnp.maximum(bv, v), bi_
        bv, bi_ = lax.fori_loop(0, nvregs, mx, (negv16, bigv))
        maxs = jnp.max(bv)
        maxv = jnp.full((16,), maxs, jnp.float32)
        bi = jnp.min(jnp.where(bv == maxv, bi_, bigv))
        biv = jnp.full((16,), bi, jnp.int32)

        def cl(j, _2):
            v = src_v[pl.ds(j * 16, 16)]
            ii = src_i[pl.ds(j * 16, 16)]
            src_v[pl.ds(j * 16, 16)] = jnp.where(
                (v == maxv) & (ii == biv), negv16, v)
            return 0
        lax.fori_loop(0, nvregs, cl, 0)
        lane0 = iota == 0
        kk = jnp.full((16,), k, jnp.int32)
        plsc.store_scatter(dst_v, [kk], jnp.full((16,), maxs, jnp.float32),
                           mask=lane0)
        plsc.store_scatter(dst_i, [kk], jnp.full((16,), bi, jnp.int32),
                           mask=lane0)
        return 0
    lax.fori_loop(0, K, body, 0)


def _sc_select_gather(object_mask, featsv):
    mesh = plsc.VectorSubcoreMesh(core_axis_name="c", subcore_axis_name="s",
                                  num_cores=2, num_subcores=16)

    @functools.partial(
        pl.kernel,
        out_type=[
            jax.ShapeDtypeStruct((B, KP, C), jnp.float32),
            jax.ShapeDtypeStruct((B, KP), jnp.float32),
        ],
        mesh=mesh,
        compiler_params=pltpu.CompilerParams(needs_layout_passes=False),
        scratch_types=[
            pltpu.VMEM((HCH, W), jnp.float32),
            pltpu.VMEM((CH + 16,), jnp.float32),
            pltpu.VMEM((CH + 16,), jnp.int32),
            pltpu.VMEM((KP,), jnp.float32),
            pltpu.VMEM((KP,), jnp.int32),
            pltpu.VMEM((4 * KP,), jnp.float32),
            pltpu.VMEM((4 * KP,), jnp.int32),
            pltpu.VMEM((KP,), jnp.float32),
            pltpu.VMEM((KP,), jnp.int32),
            pltpu.VMEM((KP,), jnp.float32),
            pltpu.VMEM((KP,), jnp.int32),
            pltpu.VMEM((16,), jnp.int32),
            pltpu.VMEM((16 * 16,), jnp.int32),
            pltpu.VMEM((KP, C), jnp.float32),
            pltpu.VMEM_SHARED((16 * 16,), jnp.int32),
            pltpu.VMEM_SHARED((16 * KP,), jnp.float32),
            pltpu.VMEM_SHARED((16 * KP,), jnp.int32),
            pltpu.SemaphoreType.DMA,
        ],
    )
    def k(mask_hbm, feat_hbm, gout_hbm, ex_hbm,
          chunk, cand_v, cand_i, loc_v, loc_i, grp_v, grp_i, mrg_v, mrg_i,
          exv, ridx, cnt_v, counts_all, rows_v,
          s_counts, s_loc_v, s_loc_i, sem):
        cid = lax.axis_index("c")
        sid = lax.axis_index("s")
        bb = sid // 4
        q = sid % 4
        b = cid * 4 + bb
        iota = lax.iota(jnp.int32, 16)

        t0v = jnp.full((16,), T0, jnp.float32)
        t1v = jnp.full((16,), T1, jnp.float32)
        one = jnp.full((16,), 1, jnp.int32)
        zero = jnp.zeros((16,), jnp.int32)
        NBLK = HCH // 8
        _ns = jax.named_scope
        ns1 = _ns("ph1_fused_scan"); ns1.__enter__()
        copies = [
            pltpu.async_copy(
                mask_hbm.at[b, 0, pl.ds(q * HCH + blk * 8, 8), :],
                chunk.at[pl.ds(blk * 8, 8), :], sem)
            for blk in range(NBLK)
        ]

        def fbody(hh, cc):
            off_v, a1 = cc
            basev = jnp.full((16,), q * CH + hh * W, jnp.int32) + iota
            for wv in range(WV):
                v = chunk[hh, pl.ds(wv * 16, 16)]
                a1 = a1 + jnp.where(v > t1v, one, zero)
                m = v > t0v
                plsc.store_compressed(cand_i.at[pl.ds(off_v[0], 16)],
                                      basev + wv * 16, mask=m)
                off_v = off_v + plsc.all_reduce_population_count(m)
            return (off_v, a1)
        acc = (zero, zero)
        for blk in range(NBLK):
            copies[blk].wait()
            acc = lax.fori_loop(blk * 8, blk * 8 + 8, fbody, acc)
        off_v, a1 = acc
        n0 = off_v[0]
        c0v = jnp.full((16,), n0, jnp.int32)
        c1v = jnp.full((16,), jnp.sum(a1), jnp.int32)
        cv = jnp.where(iota == 0, c0v, jnp.where(iota == 1, c1v,
                                                 jnp.zeros((16,), jnp.int32)))
        cnt_v[...] = cv
        pltpu.sync_copy(cnt_v, s_counts.at[pl.ds(sid * 16, 16)])
        plsc.subcore_barrier()

        pltpu.sync_copy(s_counts, counts_all)
        g0 = jnp.int32(0)
        g1 = jnp.int32(0)
        for t in range(4):
            cvec = counts_all[pl.ds((bb * 4 + t) * 16, 16)]
            g0 = g0 + jnp.minimum(cvec[0], K)
            g1 = g1 + jnp.minimum(cvec[1], K)
        ns1.__exit__(None, None, None)
        ns2 = _ns("ph2_fallback_fill"); ns2.__enter__()

        tfb = jnp.full((16,), jnp.where(g1 >= K, T1, -1.0), jnp.float32)

        def _refill(_):
            def rbody(hh, off_v2):
                basev = jnp.full((16,), q * CH + hh * W, jnp.int32) + iota
                for wv in range(WV):
                    v = chunk[hh, pl.ds(wv * 16, 16)]
                    m = v > tfb
                    plsc.store_compressed(cand_i.at[pl.ds(off_v2[0], 16)],
                                          basev + wv * 16, mask=m)
                    off_v2 = off_v2 + plsc.all_reduce_population_count(m)
                return off_v2
            return lax.fori_loop(0, HCH, rbody, zero)[0]
        n = lax.cond(g0 < K, _refill, lambda _: n0, 0)

        nv16 = jnp.full((16,), n, jnp.int32) + iota
        plsc.store_scatter(cand_i, [nv16], jnp.full((16,), q * CH, jnp.int32))
        wfull = jnp.full((16,), W, jnp.int32)
        qchv = jnp.full((16,), q * CH, jnp.int32)

        def gbody(j, _):
            li = cand_i[pl.ds(j * 16, 16)] - qchv
            cand_v[pl.ds(j * 16, 16)] = plsc.load_gather(
                chunk, [li // wfull, li % wfull])
            return 0
        lax.fori_loop(0, (n + 16) // 16, gbody, 0)
        plsc.store_scatter(cand_v, [nv16], jnp.full((16,), NEG, jnp.float32))

        ns2.__exit__(None, None, None)
        ns3 = _ns("ph3_local_extract"); ns3.__enter__()
        negv = jnp.full((16,), NEG, jnp.float32)
        zi = jnp.zeros((16,), jnp.int32)
        loc_v[pl.ds(0, 16)] = negv
        loc_v[pl.ds(16, 16)] = negv
        loc_i[pl.ds(0, 16)] = zi
        loc_i[pl.ds(16, 16)] = zi
        _extract_topk(cand_v, cand_i, loc_v, loc_i, (n + 15) // 16)

        ns3.__exit__(None, None, None)
        ns4 = _ns("ph4_publish_merge"); ns4.__enter__()
        pltpu.sync_copy(loc_v, s_loc_v.at[pl.ds(sid * KP, KP)])
        pltpu.sync_copy(loc_i, s_loc_i.at[pl.ds(sid * KP, KP)])
        plsc.subcore_barrier()

        @pl.when(q == 0)
        def _merge():
            pltpu.sync_copy(s_loc_v.at[pl.ds(bb * 4 * KP, 4 * KP)], grp_v)
            pltpu.sync_copy(s_loc_i.at[pl.ds(bb * 4 * KP, 4 * KP)], grp_i)
            mrg_v[pl.ds(0, 16)] = negv
            mrg_v[pl.ds(16, 16)] = negv
            mrg_i[pl.ds(0, 16)] = zi
            mrg_i[pl.ds(16, 16)] = zi
            _extract_topk(grp_v, grp_i, mrg_v, mrg_i, (4 * KP) // 16)
            exv[pl.ds(0, 16)] = (mrg_v[pl.ds(0, 16)] > 0.5).astype(jnp.float32)
            exv[pl.ds(16, 16)] = (mrg_v[pl.ds(16, 16)] > 0.5).astype(jnp.float32)
            pltpu.sync_copy(exv, ex_hbm.at[b])
            base = jnp.full((16,), b * HW, jnp.int32)
            ridx[pl.ds(0, 16)] = base + mrg_i[pl.ds(0, 16)]
            ridx[pl.ds(16, 16)] = base + mrg_i[pl.ds(16, 16)]
            with _ns("ph5_gather"):
                pltpu.async_copy(feat_hbm.at[ridx], rows_v, sem).wait()
                pltpu.sync_copy(rows_v, gout_hbm.at[b])

        ns4.__exit__(None, None, None)

    return k(object_mask, featsv)


def _tc_project(x, w1t, b1r, w2t, b2r):
    def mm(x_ref, w1_ref, b1_ref, w2_ref, b2_ref, o_ref):
        h = jnp.dot(x_ref[...], w1_ref[...],
                    preferred_element_type=jnp.float32) + b1_ref[...]
        h = jnp.maximum(h, 0.0)
        o_ref[...] = jnp.dot(h, w2_ref[...],
                             preferred_element_type=jnp.float32) + b2_ref[...]
    return pl.pallas_call(
        mm,
        out_shape=jax.ShapeDtypeStruct((B * KP, HID), jnp.float32),
    )(x, w1t, b1r, w2t, b2r)


def kernel(features, object_mask, W1, b1, W2, b2):
    featsv = features.transpose(0, 2, 3, 1).reshape(B * HW, C)
    gathered, exists = _sc_select_gather(object_mask, featsv)
    proj = _tc_project(gathered.reshape(B * KP, C), W1.T, b1[None, :],
                       W2.T, b2[None, :])
    selected = proj.reshape(B, KP, HID)[:, :K, :]
    return (selected, exists[:, :K])


# speedup vs baseline: 1.0478x; 1.0478x over previous
import functools

import jax
import jax.numpy as jnp
from jax import lax
from jax.experimental import pallas as pl
from jax.experimental.pallas import tpu as pltpu
from jax.experimental.pallas import tpu_sc as plsc

B, C, H, W = 8, 128, 224, 224
HW = H * W
HID = 64
K = 30
KP = 32
HCH = H // 4
CH = HW // 4
WV = W // 16
T0 = 0.995
T1 = 0.5
NEG = -2.0
BIGI = 1 << 30
CG = C // 4


def _extract_topk(src_v, src_i, dst_v, dst_i, nvregs):
    iota = lax.iota(jnp.int32, 16)
    bigv = jnp.full((16,), BIGI, jnp.int32)
    negv16 = jnp.full((16,), NEG, jnp.float32)

    def body(k, _):
        def mx(j, carry):
            bv, bi_ = carry
            v = src_v[pl.ds(j * 16, 16)]
            ii = src_i[pl.ds(j * 16, 16)]
            gt = v > bv
            eq = v == bv
            bi_ = jnp.where(gt, ii, jnp.where(eq, jnp.minimum(bi_, ii), bi_))
            return jnp.maximum(bv, v), bi_
        bv, bi_ = lax.fori_loop(0, nvregs, mx, (negv16, bigv))
        maxs = jnp.max(bv)
        maxv = jnp.full((16,), maxs, jnp.float32)
        bi = jnp.min(jnp.where(bv == maxv, bi_, bigv))
        biv = jnp.full((16,), bi, jnp.int32)

        def cl(j, _2):
            v = src_v[pl.ds(j * 16, 16)]
            ii = src_i[pl.ds(j * 16, 16)]
            src_v[pl.ds(j * 16, 16)] = jnp.where(
                (v == maxv) & (ii == biv), negv16, v)
            return 0
        lax.fori_loop(0, nvregs, cl, 0)
        lane0 = iota == 0
        kk = jnp.full((16,), k, jnp.int32)
        plsc.store_scatter(dst_v, [kk], jnp.full((16,), maxs, jnp.float32),
                           mask=lane0)
        plsc.store_scatter(dst_i, [kk], jnp.full((16,), bi, jnp.int32),
                           mask=lane0)
        return 0
    lax.fori_loop(0, K, body, 0)


def _sc_select_gather(object_mask, featsv):
    mesh = plsc.VectorSubcoreMesh(core_axis_name="c", subcore_axis_name="s",
                                  num_cores=2, num_subcores=16)

    @functools.partial(
        pl.kernel,
        out_type=[
            jax.ShapeDtypeStruct((B, KP, C), jnp.float32),
            jax.ShapeDtypeStruct((B, KP), jnp.float32),
        ],
        mesh=mesh,
        compiler_params=pltpu.CompilerParams(needs_layout_passes=False),
        scratch_types=[
            pltpu.VMEM((HCH, W), jnp.float32),
            pltpu.VMEM((CH + 16,), jnp.float32),
            pltpu.VMEM((CH + 16,), jnp.int32),
            pltpu.VMEM((KP,), jnp.float32),
            pltpu.VMEM((KP,), jnp.int32),
            pltpu.VMEM((4 * KP,), jnp.float32),
            pltpu.VMEM((4 * KP,), jnp.int32),
            pltpu.VMEM((KP,), jnp.float32),
            pltpu.VMEM((KP,), jnp.int32),
            pltpu.VMEM((KP,), jnp.float32),
            pltpu.VMEM((KP,), jnp.int32),
            pltpu.VMEM((16,), jnp.int32),
            pltpu.VMEM((16 * 16,), jnp.int32),
            pltpu.VMEM((KP, C), jnp.float32),
            pltpu.VMEM_SHARED((16 * 16,), jnp.int32),
            pltpu.VMEM_SHARED((16 * KP,), jnp.float32),
            pltpu.VMEM_SHARED((16 * KP,), jnp.int32),
            pltpu.SemaphoreType.DMA,
        ],
    )
    def k(mask_hbm, feat_hbm, gout_hbm, ex_hbm,
          chunk, cand_v, cand_i, loc_v, loc_i, grp_v, grp_i, mrg_v, mrg_i,
          exv, ridx, cnt_v, counts_all, rows_v,
          s_counts, s_loc_v, s_loc_i, sem):
        cid = lax.axis_index("c")
        sid = lax.axis_index("s")
        bb = sid // 4
        q = sid % 4
        b = cid * 4 + bb
        iota = lax.iota(jnp.int32, 16)

        t0v = jnp.full((16,), T0, jnp.float32)
        t1v = jnp.full((16,), T1, jnp.float32)
        one = jnp.full((16,), 1, jnp.int32)
        zero = jnp.zeros((16,), jnp.int32)
        NBLK = HCH // 8
        copies = [
            pltpu.async_copy(
                mask_hbm.at[b, 0, pl.ds(q * HCH + blk * 8, 8), :],
                chunk.at[pl.ds(blk * 8, 8), :], sem)
            for blk in range(NBLK)
        ]
        def cbody(hh, cc):
            a0, a1 = cc
            for wv in range(WV):
                v = chunk[hh, pl.ds(wv * 16, 16)]
                a0 = a0 + jnp.where(v > t0v, one, zero)
                a1 = a1 + jnp.where(v > t1v, one, zero)
            return (a0, a1)
        acc = (zero, zero)
        for blk in range(NBLK):
            copies[blk].wait()
            acc = lax.fori_loop(blk * 8, blk * 8 + 8, cbody, acc)
        a0, a1 = acc
        c0v = jnp.full((16,), jnp.sum(a0), jnp.int32)
        c1v = jnp.full((16,), jnp.sum(a1), jnp.int32)
        cv = jnp.where(iota == 0, c0v, jnp.where(iota == 1, c1v,
                                                 jnp.zeros((16,), jnp.int32)))
        cnt_v[...] = cv
        pltpu.sync_copy(cnt_v, s_counts.at[pl.ds(sid * 16, 16)])
        plsc.subcore_barrier()

        pltpu.sync_copy(s_counts, counts_all)
        g0 = jnp.int32(0)
        g1 = jnp.int32(0)
        for t in range(4):
            cvec = counts_all[pl.ds((bb * 4 + t) * 16, 16)]
            g0 = g0 + jnp.minimum(cvec[0], K)
            g1 = g1 + jnp.minimum(cvec[1], K)
        T = jnp.where(g0 >= K, T0, jnp.where(g1 >= K, T1, -1.0))

        tv = jnp.full((16,), T, jnp.float32)

        def pbody(hh, off_v):
            basev = jnp.full((16,), q * CH + hh * W, jnp.int32) + iota
            for wv in range(WV):
                v = chunk[hh, pl.ds(wv * 16, 16)]
                m = v > tv
                off_s = off_v[0]
                plsc.store_compressed(cand_v.at[pl.ds(off_s, 16)], v, mask=m)
                plsc.store_compressed(cand_i.at[pl.ds(off_s, 16)],
                                      basev + wv * 16, mask=m)
                off_v = off_v + plsc.all_reduce_population_count(m)
            return off_v
        n = lax.fori_loop(0, HCH, pbody, jnp.zeros((16,), jnp.int32))[0]
        nv16 = jnp.full((16,), n, jnp.int32) + iota
        plsc.store_scatter(cand_v, [nv16], jnp.full((16,), NEG, jnp.float32))
        plsc.store_scatter(cand_i, [nv16], jnp.zeros((16,), jnp.int32))

        negv = jnp.full((16,), NEG, jnp.float32)
        zi = jnp.zeros((16,), jnp.int32)
        loc_v[pl.ds(0, 16)] = negv
        loc_v[pl.ds(16, 16)] = negv
        loc_i[pl.ds(0, 16)] = zi
        loc_i[pl.ds(16, 16)] = zi
        _extract_topk(cand_v, cand_i, loc_v, loc_i, (n + 15) // 16)

        pltpu.sync_copy(loc_v, s_loc_v.at[pl.ds(sid * KP, KP)])
        pltpu.sync_copy(loc_i, s_loc_i.at[pl.ds(sid * KP, KP)])
        plsc.subcore_barrier()

        @pl.when(q == 0)
        def _merge():
            pltpu.sync_copy(s_loc_v.at[pl.ds(bb * 4 * KP, 4 * KP)], grp_v)
            pltpu.sync_copy(s_loc_i.at[pl.ds(bb * 4 * KP, 4 * KP)], grp_i)
            mrg_v[pl.ds(0, 16)] = negv
            mrg_v[pl.ds(16, 16)] = negv
            mrg_i[pl.ds(0, 16)] = zi
            mrg_i[pl.ds(16, 16)] = zi
            _extract_topk(grp_v, grp_i, mrg_v, mrg_i, (4 * KP) // 16)
            exv[pl.ds(0, 16)] = (mrg_v[pl.ds(0, 16)] > 0.5).astype(jnp.float32)
            exv[pl.ds(16, 16)] = (mrg_v[pl.ds(16, 16)] > 0.5).astype(jnp.float32)
            pltpu.sync_copy(exv, ex_hbm.at[b])
            base = jnp.full((16,), b * HW, jnp.int32)
            ridx[pl.ds(0, 16)] = base + mrg_i[pl.ds(0, 16)]
            ridx[pl.ds(16, 16)] = base + mrg_i[pl.ds(16, 16)]
            pltpu.async_copy(feat_hbm.at[ridx], rows_v, sem).wait()
            pltpu.sync_copy(rows_v, gout_hbm.at[b])

    return k(object_mask, featsv)


def _tc_project(x, w1t, b1r, w2t, b2r):
    def mm(x_ref, w1_ref, b1_ref, w2_ref, b2_ref, o_ref):
        h = jnp.dot(x_ref[...], w1_ref[...],
                    preferred_element_type=jnp.float32) + b1_ref[...]
        h = jnp.maximum(h, 0.0)
        o_ref[...] = jnp.dot(h, w2_ref[...],
                             preferred_element_type=jnp.float32) + b2_ref[...]
    return pl.pallas_call(
        mm,
        out_shape=jax.ShapeDtypeStruct((B * KP, HID), jnp.float32),
    )(x, w1t, b1r, w2t, b2r)


def kernel(features, object_mask, W1, b1, W2, b2):
    featsv = features.transpose(0, 2, 3, 1).reshape(B * HW, C)
    gathered, exists = _sc_select_gather(object_mask, featsv)
    proj = _tc_project(gathered.reshape(B * KP, C), W1.T, b1[None, :],
                       W2.T, b2[None, :])
    selected = proj.reshape(B, KP, HID)[:, :K, :]
    return (selected, exists[:, :K])
